# Initial kernel scaffold; baseline (speedup 1.0000x reference)
import functools

import jax
import jax.numpy as jnp
from jax import lax
from jax.experimental import pallas as pl
from jax.experimental.pallas import tpu as pltpu
from jax.experimental.pallas import tpu_sc as plsc

B = 16384
V = 100000
D = 16
NT = 26
NS = 2
L = 50
LP = 64
INNER = 128

NW = 32
CB = B // NW
C = 16
NCHUNK = CB // C
TOKW = C * NT
SEQW = C * NS * LP


def _sc_stage():
    mesh = plsc.VectorSubcoreMesh(core_axis_name="c", subcore_axis_name="s")

    @functools.partial(
        pl.kernel,
        mesh=mesh,
        out_type=[
            jax.ShapeDtypeStruct((B * NT, D), jnp.float32),
            jax.ShapeDtypeStruct((NS * B, D), jnp.float32),
            jax.ShapeDtypeStruct((B,), jnp.float32),
            jax.ShapeDtypeStruct((NS * B,), jnp.float32),
            jax.ShapeDtypeStruct((NS * B,), jnp.float32),
        ],
        scratch_types=[
            pltpu.VMEM((TOKW,), jnp.int32),
            pltpu.VMEM((TOKW,), jnp.int32),
            pltpu.VMEM((SEQW,), jnp.int32),
            pltpu.VMEM((TOKW, D), jnp.float32),
            pltpu.VMEM((SEQW, D), jnp.float32),
            pltpu.VMEM((TOKW,), jnp.float32),
            pltpu.VMEM((SEQW,), jnp.float32),
            pltpu.VMEM((NS * C, D), jnp.float32),
            pltpu.VMEM((C,), jnp.float32),
            pltpu.VMEM((NS * C,), jnp.float32),
            pltpu.VMEM((NS * C,), jnp.float32),
            pltpu.SemaphoreType.DMA,
            pltpu.SemaphoreType.DMA,
            pltpu.SemaphoreType.DMA,
            pltpu.SemaphoreType.DMA,
        ],
    )
    def k(tok_idx_h, tokfo_idx_h, seq_idx_h, so_tok_h, so_seq_h, fo_tok_h,
          fo_seq_h, emb_tok_h, seq_sum_h, tok_fo_h, seq_fo_h, cnt_h,
          tok_idx_v, tokfo_idx_v, seq_idx_v, tok_rows, seq_rows, tok_fo_v,
          seq_fo_v, seq_sum_v, tok_fo_sv, seq_fo_sv, cnt_sv,
          sem0, sem1, sem2, sem3):
        wid = lax.axis_index("s") * 2 + lax.axis_index("c")

        def chunk_body(i, carry):
            g = wid * NCHUNK + i
            tok_base = g * TOKW
            seq_base = g * SEQW
            b_base = g * C

            pltpu.sync_copy(tok_idx_h.at[pl.ds(tok_base, TOKW)], tok_idx_v)
            pltpu.sync_copy(tokfo_idx_h.at[pl.ds(tok_base, TOKW)], tokfo_idx_v)
            pltpu.sync_copy(seq_idx_h.at[pl.ds(seq_base, SEQW)], seq_idx_v)

            c0 = pltpu.async_copy(so_tok_h.at[tok_idx_v], tok_rows, sem0)
            c1 = pltpu.async_copy(so_seq_h.at[seq_idx_v], seq_rows, sem1)
            c2 = pltpu.async_copy(fo_tok_h.at[tokfo_idx_v], tok_fo_v, sem2)
            c3 = pltpu.async_copy(fo_seq_h.at[seq_idx_v], seq_fo_v, sem3)
            c0.wait()
            c2.wait()

            pltpu.sync_copy(tok_rows, emb_tok_h.at[pl.ds(tok_base, TOKW)])

            acc = tok_fo_v[pl.ds(0, C)]
            for f in range(1, NT):
                acc = acc + tok_fo_v[pl.ds(f * C, C)]
            tok_fo_sv[...] = acc

            c3.wait()
            for s in range(NS):
                sent = jnp.int32(s * V)
                facc = seq_fo_v[pl.ds((s * LP) * C, C)]
                idx0 = seq_idx_v[pl.ds((s * LP) * C, C)]
                cacc = jnp.where(idx0 != sent, 1.0, 0.0).astype(jnp.float32)
                for j in range(1, LP):
                    off = (s * LP + j) * C
                    facc = facc + seq_fo_v[pl.ds(off, C)]
                    idxj = seq_idx_v[pl.ds(off, C)]
                    cacc = cacc + jnp.where(idxj != sent, 1.0, 0.0).astype(jnp.float32)
                seq_fo_sv[pl.ds(s * C, C)] = facc
                cnt_sv[pl.ds(s * C, C)] = cacc

            c1.wait()
            for s in range(NS):
                def jstep(j, accs):
                    base = (s * LP + j) * C
                    return tuple(accs[b] + seq_rows[base + b] for b in range(C))
                accs = tuple(seq_rows[(s * LP) * C + b] for b in range(C))
                accs = lax.fori_loop(1, LP, jstep, accs)
                for b in range(C):
                    seq_sum_v[s * C + b] = accs[b]

            pltpu.sync_copy(tok_fo_sv, tok_fo_h.at[pl.ds(b_base, C)])
            for s in range(NS):
                pltpu.sync_copy(seq_fo_sv.at[pl.ds(s * C, C)],
                                seq_fo_h.at[pl.ds(s * B + b_base, C)])
                pltpu.sync_copy(cnt_sv.at[pl.ds(s * C, C)],
                                cnt_h.at[pl.ds(s * B + b_base, C)])
                pltpu.sync_copy(seq_sum_v.at[pl.ds(s * C, C)],
                                seq_sum_h.at[pl.ds(s * B + b_base, C)])
            return carry

        lax.fori_loop(0, NCHUNK, chunk_body, 0)

    return k


def _tc_stage(bb):
    grid = (B // bb,)

    def body(emb_ref, ssum_ref, cnt_ref, sfo_ref, tfo_ref, lab_ref, row0_ref,
             fo0_ref, w1_ref, b1_ref, w2_ref, b2_ref, bias_ref, out_ref):
        i = pl.program_id(0)
        et = emb_ref[...]
        ssum = ssum_ref[...]
        cnt = cnt_ref[...]
        sfo = sfo_ref[...]
        row0 = row0_ref[...]
        fo0 = fo0_ref[...]
        npad = jnp.float32(LP) - cnt
        denom = jnp.maximum(cnt, 1.0)
        sagg = (ssum - npad[:, :, None] * row0[:, None, :]) / denom[:, :, None]
        sfo_c = (sfo - npad * fo0.reshape(NS, 1)) / denom
        first = tfo_ref[...][0] + sfo_c.sum(axis=0)

        et3 = et.reshape(bb, NT, D)
        summed = et3.sum(axis=1) + sagg.sum(axis=0)
        sumsq = (et3 * et3).sum(axis=1) + (sagg * sagg).sum(axis=0)
        second = 0.5 * ((summed * summed - sumsq).sum(axis=-1))

        w1 = w1_ref[...]
        h = jnp.dot(et, w1[: NT * D], preferred_element_type=jnp.float32)
        for s in range(NS):
            h = h + jnp.dot(sagg[s], w1[NT * D + s * D: NT * D + (s + 1) * D],
                            preferred_element_type=jnp.float32)
        h = jax.nn.relu(h + b1_ref[...])
        dnn = jnp.dot(h, w2_ref[...], preferred_element_type=jnp.float32)[:, 0]
        dnn = dnn + b2_ref[0, 0]

        logits = bias_ref[0, 0] + first + second + dnn
        y = lab_ref[...][0].astype(jnp.float32)
        bce = (jnp.maximum(logits, 0.0) - logits * y
               + jnp.log1p(jnp.exp(-jnp.abs(logits))))
        part = bce.sum() * (1.0 / B)

        @pl.when(i == 0)
        def _init():
            out_ref[0, 0] = 0.0

        out_ref[0, 0] += part

    return pl.pallas_call(
        body,
        grid=grid,
        in_specs=[
            pl.BlockSpec((bb, NT * D), lambda i: (i, 0)),
            pl.BlockSpec((NS, bb, D), lambda i: (0, i, 0)),
            pl.BlockSpec((NS, bb), lambda i: (0, i)),
            pl.BlockSpec((NS, bb), lambda i: (0, i)),
            pl.BlockSpec((1, bb), lambda i: (0, i)),
            pl.BlockSpec((1, bb), lambda i: (0, i)),
            pl.BlockSpec((NS, D), lambda i: (0, 0)),
            pl.BlockSpec((1, NS), lambda i: (0, 0)),
            pl.BlockSpec(((NT + NS) * D, INNER), lambda i: (0, 0)),
            pl.BlockSpec((1, INNER), lambda i: (0, 0)),
            pl.BlockSpec((INNER, 1), lambda i: (0, 0)),
            pl.BlockSpec((1, 1), lambda i: (0, 0)),
            pl.BlockSpec((1, 1), lambda i: (0, 0)),
        ],
        out_specs=pl.BlockSpec((1, 1), lambda i: (0, 0)),
        out_shape=jax.ShapeDtypeStruct((1, 1), jnp.float32),
    )


def kernel(token_field_values, token_sequence_field_values, labels, fo_token,
           so_token, fo_seq, so_seq, fm_bias, W1, b1, W2, b2):
    tfv = token_field_values.astype(jnp.int32)
    tok_flat = tfv + (jnp.arange(NT, dtype=jnp.int32) * V)[None, :]
    tok_idx = tok_flat.reshape(-1)
    tok_fo_idx = tok_flat.reshape(B // C, C, NT).transpose(0, 2, 1).reshape(-1)

    sq = token_sequence_field_values.astype(jnp.int32)
    sqp = jnp.pad(sq, ((0, 0), (0, 0), (0, LP - L)))
    seq_flat = sqp + (jnp.arange(NS, dtype=jnp.int32) * V)[None, :, None]
    seq_idx = seq_flat.reshape(B // C, C, NS * LP).transpose(0, 2, 1).reshape(-1)

    so_tok_flat = so_token.reshape(NT * V, D)
    so_seq_flat = so_seq.reshape(NS * V, D)
    fo_tok_flat = fo_token.reshape(NT * V)
    fo_seq_flat = fo_seq.reshape(NS * V)

    emb_tok, seq_sum, tok_fo, seq_fo, cnt = _sc_stage()(
        tok_idx, tok_fo_idx, seq_idx, so_tok_flat, so_seq_flat,
        fo_tok_flat, fo_seq_flat)

    bb = 2048
    out = _tc_stage(bb)(
        emb_tok.reshape(B, NT * D),
        seq_sum.reshape(NS, B, D),
        cnt.reshape(NS, B),
        seq_fo.reshape(NS, B),
        tok_fo.reshape(1, B),
        labels.astype(jnp.int32).reshape(1, B),
        so_seq[:, 0, :],
        fo_seq[:, 0].reshape(1, NS),
        W1, b1.reshape(1, INNER), W2, b2.reshape(1, 1),
        fm_bias.reshape(1, 1),
    )
    return out[0, 0]


# speedup vs baseline: 16.9321x; 16.9321x over previous
import functools

import jax
import jax.numpy as jnp
from jax import lax
from jax.experimental import pallas as pl
from jax.experimental.pallas import tpu as pltpu
from jax.experimental.pallas import tpu_sc as plsc

B = 16384
V = 100000
D = 16
NT = 26
NS = 2
L = 50
LP = 64
INNER = 128

NW = 32
CB = B // NW
C = 16
NCHUNK = CB // C
TOKW = C * NT
SEQW = C * NS * LP


def _sc_stage():
    mesh = plsc.VectorSubcoreMesh(core_axis_name="c", subcore_axis_name="s")

    @functools.partial(
        pl.kernel,
        mesh=mesh,
        compiler_params=pltpu.CompilerParams(use_tc_tiling_on_sc=False),
        out_type=[
            jax.ShapeDtypeStruct((B * NT, D), jnp.float32),
            jax.ShapeDtypeStruct((NS * B, D), jnp.float32),
            jax.ShapeDtypeStruct((B,), jnp.float32),
            jax.ShapeDtypeStruct((NS * B,), jnp.float32),
            jax.ShapeDtypeStruct((NS * B,), jnp.float32),
        ],
        scratch_types=[
            pltpu.VMEM((TOKW,), jnp.int32),
            pltpu.VMEM((TOKW,), jnp.int32),
            pltpu.VMEM((SEQW,), jnp.int32),
            pltpu.VMEM((TOKW, D), jnp.float32),
            pltpu.VMEM((SEQW, D), jnp.float32),
            pltpu.VMEM((TOKW,), jnp.float32),
            pltpu.VMEM((SEQW,), jnp.float32),
            pltpu.VMEM((NS * C, D), jnp.float32),
            pltpu.VMEM((C,), jnp.float32),
            pltpu.VMEM((NS * C,), jnp.float32),
            pltpu.VMEM((NS * C,), jnp.float32),
            pltpu.SemaphoreType.DMA,
            pltpu.SemaphoreType.DMA,
            pltpu.SemaphoreType.DMA,
            pltpu.SemaphoreType.DMA,
        ],
    )
    def k(tok_idx_h, tokfo_idx_h, seq_idx_h, so_tok_h, so_seq_h, fo_tok_h,
          fo_seq_h, emb_tok_h, seq_sum_h, tok_fo_h, seq_fo_h, cnt_h,
          tok_idx_v, tokfo_idx_v, seq_idx_v, tok_rows, seq_rows, tok_fo_v,
          seq_fo_v, seq_sum_v, tok_fo_sv, seq_fo_sv, cnt_sv,
          sem0, sem1, sem2, sem3):
        wid = lax.axis_index("s") * 2 + lax.axis_index("c")

        def chunk_body(i, carry):
            g = wid * NCHUNK + i
            tok_base = g * TOKW
            seq_base = g * SEQW
            b_base = g * C

            pltpu.sync_copy(tok_idx_h.at[pl.ds(tok_base, TOKW)], tok_idx_v)
            pltpu.sync_copy(tokfo_idx_h.at[pl.ds(tok_base, TOKW)], tokfo_idx_v)
            pltpu.sync_copy(seq_idx_h.at[pl.ds(seq_base, SEQW)], seq_idx_v)

            c0 = pltpu.async_copy(so_tok_h.at[tok_idx_v], tok_rows, sem0)
            c1 = pltpu.async_copy(so_seq_h.at[seq_idx_v], seq_rows, sem1)
            c2 = pltpu.async_copy(fo_tok_h.at[tokfo_idx_v], tok_fo_v, sem2)
            c3 = pltpu.async_copy(fo_seq_h.at[seq_idx_v], seq_fo_v, sem3)
            c0.wait()
            c2.wait()

            pltpu.sync_copy(tok_rows, emb_tok_h.at[pl.ds(tok_base, TOKW)])

            acc = tok_fo_v[pl.ds(0, C)]
            for f in range(1, NT):
                acc = acc + tok_fo_v[pl.ds(f * C, C)]
            tok_fo_sv[...] = acc

            c3.wait()
            for s in range(NS):
                sent = jnp.int32(s * V)
                facc = seq_fo_v[pl.ds((s * LP) * C, C)]
                idx0 = seq_idx_v[pl.ds((s * LP) * C, C)]
                cacc = jnp.where(idx0 != sent, 1.0, 0.0).astype(jnp.float32)
                for j in range(1, LP):
                    off = (s * LP + j) * C
                    facc = facc + seq_fo_v[pl.ds(off, C)]
                    idxj = seq_idx_v[pl.ds(off, C)]
                    cacc = cacc + jnp.where(idxj != sent, 1.0, 0.0).astype(jnp.float32)
                seq_fo_sv[pl.ds(s * C, C)] = facc
                cnt_sv[pl.ds(s * C, C)] = cacc

            c1.wait()
            for s in range(NS):
                def jstep(j, accs):
                    base = (s * LP + j) * C
                    return tuple(accs[b] + seq_rows[base + b] for b in range(C))
                accs = tuple(seq_rows[(s * LP) * C + b] for b in range(C))
                accs = lax.fori_loop(1, LP, jstep, accs)
                for b in range(C):
                    seq_sum_v[s * C + b] = accs[b]

            pltpu.sync_copy(tok_fo_sv, tok_fo_h.at[pl.ds(b_base, C)])
            for s in range(NS):
                pltpu.sync_copy(seq_fo_sv.at[pl.ds(s * C, C)],
                                seq_fo_h.at[pl.ds(s * B + b_base, C)])
                pltpu.sync_copy(cnt_sv.at[pl.ds(s * C, C)],
                                cnt_h.at[pl.ds(s * B + b_base, C)])
                pltpu.sync_copy(seq_sum_v.at[pl.ds(s * C, C)],
                                seq_sum_h.at[pl.ds(s * B + b_base, C)])
            return carry

        lax.fori_loop(0, NCHUNK, chunk_body, 0)

    return k


def _tc_stage(bb):
    grid = (B // bb,)

    def body(emb_ref, ssum_ref, cnt_ref, sfo_ref, tfo_ref, lab_ref, row0_ref,
             fo0_ref, w1_ref, b1_ref, w2_ref, b2_ref, bias_ref, out_ref):
        i = pl.program_id(0)
        et = emb_ref[...]
        ssum = ssum_ref[...]
        cnt = cnt_ref[...]
        sfo = sfo_ref[...]
        row0 = row0_ref[...]
        fo0 = fo0_ref[...]
        npad = jnp.float32(LP) - cnt
        denom = jnp.maximum(cnt, 1.0)
        sagg = (ssum - npad[:, :, None] * row0[:, None, :]) / denom[:, :, None]
        sfo_c = (sfo - npad * fo0.reshape(NS, 1)) / denom
        first = tfo_ref[...][0] + sfo_c.sum(axis=0)

        et3 = et.reshape(bb, NT, D)
        summed = et3.sum(axis=1) + sagg.sum(axis=0)
        sumsq = (et3 * et3).sum(axis=1) + (sagg * sagg).sum(axis=0)
        second = 0.5 * ((summed * summed - sumsq).sum(axis=-1))

        w1 = w1_ref[...]
        h = jnp.dot(et, w1[: NT * D], preferred_element_type=jnp.float32)
        for s in range(NS):
            h = h + jnp.dot(sagg[s], w1[NT * D + s * D: NT * D + (s + 1) * D],
                            preferred_element_type=jnp.float32)
        h = jax.nn.relu(h + b1_ref[...])
        dnn = jnp.dot(h, w2_ref[...], preferred_element_type=jnp.float32)[:, 0]
        dnn = dnn + b2_ref[0, 0]

        logits = bias_ref[0, 0] + first + second + dnn
        y = lab_ref[...][0].astype(jnp.float32)
        bce = (jnp.maximum(logits, 0.0) - logits * y
               + jnp.log1p(jnp.exp(-jnp.abs(logits))))
        part = bce.sum() * (1.0 / B)

        @pl.when(i == 0)
        def _init():
            out_ref[...] = jnp.zeros((1, 1), jnp.float32)

        out_ref[...] += part.reshape(1, 1)

    return pl.pallas_call(
        body,
        grid=grid,
        in_specs=[
            pl.BlockSpec((bb, NT * D), lambda i: (i, 0)),
            pl.BlockSpec((NS, bb, D), lambda i: (0, i, 0)),
            pl.BlockSpec((NS, bb), lambda i: (0, i)),
            pl.BlockSpec((NS, bb), lambda i: (0, i)),
            pl.BlockSpec((1, bb), lambda i: (0, i)),
            pl.BlockSpec((1, bb), lambda i: (0, i)),
            pl.BlockSpec((NS, D), lambda i: (0, 0)),
            pl.BlockSpec((1, NS), lambda i: (0, 0)),
            pl.BlockSpec(((NT + NS) * D, INNER), lambda i: (0, 0)),
            pl.BlockSpec((1, INNER), lambda i: (0, 0)),
            pl.BlockSpec((INNER, 1), lambda i: (0, 0)),
            pl.BlockSpec((1, 1), lambda i: (0, 0)),
            pl.BlockSpec((1, 1), lambda i: (0, 0)),
        ],
        out_specs=pl.BlockSpec((1, 1), lambda i: (0, 0)),
        out_shape=jax.ShapeDtypeStruct((1, 1), jnp.float32),
    )


def kernel(token_field_values, token_sequence_field_values, labels, fo_token,
           so_token, fo_seq, so_seq, fm_bias, W1, b1, W2, b2):
    tfv = token_field_values.astype(jnp.int32)
    tok_flat = tfv + (jnp.arange(NT, dtype=jnp.int32) * V)[None, :]
    tok_idx = tok_flat.reshape(-1)
    tok_fo_idx = tok_flat.reshape(B // C, C, NT).transpose(0, 2, 1).reshape(-1)

    sq = token_sequence_field_values.astype(jnp.int32)
    sqp = jnp.pad(sq, ((0, 0), (0, 0), (0, LP - L)))
    seq_flat = sqp + (jnp.arange(NS, dtype=jnp.int32) * V)[None, :, None]
    seq_idx = seq_flat.reshape(B // C, C, NS * LP).transpose(0, 2, 1).reshape(-1)

    so_tok_flat = so_token.reshape(NT * V, D)
    so_seq_flat = so_seq.reshape(NS * V, D)
    fo_tok_flat = fo_token.reshape(NT * V)
    fo_seq_flat = fo_seq.reshape(NS * V)

    emb_tok, seq_sum, tok_fo, seq_fo, cnt = _sc_stage()(
        tok_idx, tok_fo_idx, seq_idx, so_tok_flat, so_seq_flat,
        fo_tok_flat, fo_seq_flat)

    bb = 2048
    out = _tc_stage(bb)(
        emb_tok.reshape(B, NT * D),
        seq_sum.reshape(NS, B, D),
        cnt.reshape(NS, B),
        seq_fo.reshape(NS, B),
        tok_fo.reshape(1, B),
        labels.astype(jnp.int32).reshape(1, B),
        so_seq[:, 0, :],
        fo_seq[:, 0].reshape(1, NS),
        W1, b1.reshape(1, INNER), W2, b2.reshape(1, 1),
        fm_bias.reshape(1, 1),
    )
    return out[0, 0]


# speedup vs baseline: 17.6483x; 1.0423x over previous
import functools

import jax
import jax.numpy as jnp
from jax import lax
from jax.experimental import pallas as pl
from jax.experimental.pallas import tpu as pltpu
from jax.experimental.pallas import tpu_sc as plsc

B = 16384
V = 100000
D = 16
NT = 26
NS = 2
L = 50
LP = 64
INNER = 128

NW = 32
CB = B // NW
C = 16
NCHUNK = CB // C
TOKW = C * NT
SEQW = C * NS * LP
HS = SEQW // NS


def _sc_stage():
    mesh = plsc.VectorSubcoreMesh(core_axis_name="c", subcore_axis_name="s")

    vm = pltpu.VMEM
    f32 = jnp.float32
    i32 = jnp.int32

    @functools.partial(
        pl.kernel,
        mesh=mesh,
        compiler_params=pltpu.CompilerParams(use_tc_tiling_on_sc=False),
        out_type=[
            jax.ShapeDtypeStruct((B * NT, D), f32),
            jax.ShapeDtypeStruct((NS * B, D), f32),
            jax.ShapeDtypeStruct((B,), f32),
            jax.ShapeDtypeStruct((NS * B,), f32),
            jax.ShapeDtypeStruct((NS * B,), f32),
        ],
        scratch_types=[
            [vm((TOKW,), i32)] * 2,
            [vm((TOKW,), i32)] * 2,
            [vm((HS,), i32)] * 2,
            [vm((HS,), i32)] * 2,
            [vm((TOKW, D), f32)] * 2,
            [vm((HS, D), f32)] * 2,
            [vm((HS, D), f32)] * 2,
            [vm((TOKW,), f32)] * 2,
            [vm((HS,), f32)] * 2,
            [vm((HS,), f32)] * 2,
            vm((NS * C, D), f32),
            vm((C,), f32),
            vm((NS * C,), f32),
            vm((NS * C,), f32),
            [pltpu.SemaphoreType.DMA] * 2,
            [pltpu.SemaphoreType.DMA] * 2,
            [pltpu.SemaphoreType.DMA] * 2,
            [pltpu.SemaphoreType.DMA] * 2,
            [pltpu.SemaphoreType.DMA] * 2,
            [pltpu.SemaphoreType.DMA] * 2,
            [pltpu.SemaphoreType.DMA] * 2,
            pltpu.SemaphoreType.DMA,
        ],
    )
    def k(tok_idx_h, tokfo_idx_h, seq_idx_h, so_tok_h, so_seq_h, fo_tok_h,
          fo_seq_h, emb_tok_h, seq_sum_h, tok_fo_h, seq_fo_h, cnt_h,
          ti, tf, sA, sB, tr, srA, srB, ftv, fsA, fsB,
          seq_sum_v, tok_fo_sv, seq_fo_sv, cnt_sv,
          isem, g_tok, g_sA, g_sB, g_ft, g_fA, g_fB, wsem):
        wid = lax.axis_index("s") * 2 + lax.axis_index("c")

        def idx_copies(gl, p):
            gg = wid * NCHUNK + gl
            return (
                pltpu.make_async_copy(
                    tok_idx_h.at[pl.ds(gg * TOKW, TOKW)], ti[p], isem[p]),
                pltpu.make_async_copy(
                    tokfo_idx_h.at[pl.ds(gg * TOKW, TOKW)], tf[p], isem[p]),
                pltpu.make_async_copy(
                    seq_idx_h.at[pl.ds(gg * SEQW, HS)], sA[p], isem[p]),
                pltpu.make_async_copy(
                    seq_idx_h.at[pl.ds(gg * SEQW + HS, HS)], sB[p], isem[p]),
            )

        def fire_idx(gl, p):
            for c in idx_copies(gl, p):
                c.start()

        def wait_idx(gl, p):
            for c in idx_copies(gl, p):
                c.wait()

        def gather_copies(p):
            return (
                pltpu.make_async_copy(so_tok_h.at[ti[p]], tr[p], g_tok[p]),
                pltpu.make_async_copy(so_seq_h.at[sA[p]], srA[p], g_sA[p]),
                pltpu.make_async_copy(so_seq_h.at[sB[p]], srB[p], g_sB[p]),
                pltpu.make_async_copy(fo_tok_h.at[tf[p]], ftv[p], g_ft[p]),
                pltpu.make_async_copy(fo_seq_h.at[sA[p]], fsA[p], g_fA[p]),
                pltpu.make_async_copy(fo_seq_h.at[sB[p]], fsB[p], g_fB[p]),
            )

        def fire_gathers(p):
            for c in gather_copies(p):
                c.start()

        def wait_gathers(p):
            for c in gather_copies(p):
                c.wait()

        def process(gl, p, k_iter, last_fire_ok):
            gg = wid * NCHUNK + gl
            tok_base = gg * TOKW
            b_base = gg * C

            wait_gathers(p)

            wcopies = [pltpu.make_async_copy(
                tr[p], emb_tok_h.at[pl.ds(tok_base, TOKW)], wsem)]
            wcopies[0].start()

            acc0 = ftv[p][pl.ds(0, C)] + ftv[p][pl.ds(C, C)]
            acc1 = ftv[p][pl.ds(2 * C, C)] + ftv[p][pl.ds(3 * C, C)]
            for f in range(4, NT, 2):
                acc0 = acc0 + ftv[p][pl.ds(f * C, C)]
                acc1 = acc1 + ftv[p][pl.ds((f + 1) * C, C)]
            tok_fo_sv[...] = acc0 + acc1

            for s, (idxb, fob) in enumerate(((sA[p], fsA[p]), (sB[p], fsB[p]))):
                sent = jnp.int32(s * V)
                f0 = fob[pl.ds(0, C)]
                f1 = fob[pl.ds(C, C)]
                c0 = jnp.where(idxb[pl.ds(0, C)] != sent, 1.0, 0.0)
                c1 = jnp.where(idxb[pl.ds(C, C)] != sent, 1.0, 0.0)
                for j in range(2, LP, 2):
                    f0 = f0 + fob[pl.ds(j * C, C)]
                    f1 = f1 + fob[pl.ds((j + 1) * C, C)]
                    c0 = c0 + jnp.where(idxb[pl.ds(j * C, C)] != sent, 1.0, 0.0)
                    c1 = c1 + jnp.where(idxb[pl.ds((j + 1) * C, C)] != sent,
                                        1.0, 0.0)
                seq_fo_sv[pl.ds(s * C, C)] = f0 + f1
                cnt_sv[pl.ds(s * C, C)] = c0 + c1

            if last_fire_ok is not None:
                @pl.when(last_fire_ok)
                def _():
                    fire_idx(gl + 2, p)

            for s, rows in enumerate((srA[p], srB[p])):
                def jstep(j, accs):
                    base = j * C
                    return tuple(accs[b] + rows[base + b] for b in range(C))
                accs = tuple(rows[b] for b in range(C))
                accs = lax.fori_loop(1, LP, jstep, accs)
                for b in range(C):
                    seq_sum_v[s * C + b] = accs[b]

            wcopies.append(pltpu.make_async_copy(
                tok_fo_sv, tok_fo_h.at[pl.ds(b_base, C)], wsem))
            for s in range(NS):
                wcopies.append(pltpu.make_async_copy(
                    seq_fo_sv.at[pl.ds(s * C, C)],
                    seq_fo_h.at[pl.ds(s * B + b_base, C)], wsem))
                wcopies.append(pltpu.make_async_copy(
                    cnt_sv.at[pl.ds(s * C, C)],
                    cnt_h.at[pl.ds(s * B + b_base, C)], wsem))
                wcopies.append(pltpu.make_async_copy(
                    seq_sum_v.at[pl.ds(s * C, C)],
                    seq_sum_h.at[pl.ds(s * B + b_base, C)], wsem))
            for c in wcopies[1:]:
                c.start()
            for c in wcopies:
                c.wait()

        fire_idx(0, 0)
        wait_idx(0, 0)
        fire_gathers(0)
        fire_idx(1, 1)

        def body(kk, carry):
            g = 2 * kk
            wait_idx(g + 1, 1)
            fire_gathers(1)
            process(g, 0, kk, kk < (NCHUNK // 2 - 1))
            ok = kk < (NCHUNK // 2 - 1)

            @pl.when(ok)
            def _():
                wait_idx(g + 2, 0)
                fire_gathers(0)

            process(g + 1, 1, kk, ok)
            return carry

        lax.fori_loop(0, NCHUNK // 2, body, 0)

    return k


def _tc_stage(bb):
    grid = (B // bb,)

    def body(emb_ref, ssum_ref, cnt_ref, sfo_ref, tfo_ref, lab_ref, row0_ref,
             fo0_ref, w1_ref, b1_ref, w2_ref, b2_ref, bias_ref, out_ref):
        i = pl.program_id(0)
        et = emb_ref[...]
        ssum = ssum_ref[...]
        cnt = cnt_ref[...]
        sfo = sfo_ref[...]
        row0 = row0_ref[...]
        fo0 = fo0_ref[...]
        npad = jnp.float32(LP) - cnt
        denom = jnp.maximum(cnt, 1.0)
        sagg = (ssum - npad[:, :, None] * row0[:, None, :]) / denom[:, :, None]
        sfo_c = (sfo - npad * fo0.reshape(NS, 1)) / denom
        first = tfo_ref[...][0] + sfo_c.sum(axis=0)

        et3 = et.reshape(bb, NT, D)
        summed = et3.sum(axis=1) + sagg.sum(axis=0)
        sumsq = (et3 * et3).sum(axis=1) + (sagg * sagg).sum(axis=0)
        second = 0.5 * ((summed * summed - sumsq).sum(axis=-1))

        w1 = w1_ref[...]
        h = jnp.dot(et, w1[: NT * D], preferred_element_type=jnp.float32)
        for s in range(NS):
            h = h + jnp.dot(sagg[s], w1[NT * D + s * D: NT * D + (s + 1) * D],
                            preferred_element_type=jnp.float32)
        h = jax.nn.relu(h + b1_ref[...])
        dnn = jnp.dot(h, w2_ref[...], preferred_element_type=jnp.float32)[:, 0]
        dnn = dnn + b2_ref[0, 0]

        logits = bias_ref[0, 0] + first + second + dnn
        y = lab_ref[...][0].astype(jnp.float32)
        bce = (jnp.maximum(logits, 0.0) - logits * y
               + jnp.log1p(jnp.exp(-jnp.abs(logits))))
        part = bce.sum() * (1.0 / B)

        @pl.when(i == 0)
        def _init():
            out_ref[...] = jnp.zeros((1, 1), jnp.float32)

        out_ref[...] += part.reshape(1, 1)

    return pl.pallas_call(
        body,
        grid=grid,
        in_specs=[
            pl.BlockSpec((bb, NT * D), lambda i: (i, 0)),
            pl.BlockSpec((NS, bb, D), lambda i: (0, i, 0)),
            pl.BlockSpec((NS, bb), lambda i: (0, i)),
            pl.BlockSpec((NS, bb), lambda i: (0, i)),
            pl.BlockSpec((1, bb), lambda i: (0, i)),
            pl.BlockSpec((1, bb), lambda i: (0, i)),
            pl.BlockSpec((NS, D), lambda i: (0, 0)),
            pl.BlockSpec((1, NS), lambda i: (0, 0)),
            pl.BlockSpec(((NT + NS) * D, INNER), lambda i: (0, 0)),
            pl.BlockSpec((1, INNER), lambda i: (0, 0)),
            pl.BlockSpec((INNER, 1), lambda i: (0, 0)),
            pl.BlockSpec((1, 1), lambda i: (0, 0)),
            pl.BlockSpec((1, 1), lambda i: (0, 0)),
        ],
        out_specs=pl.BlockSpec((1, 1), lambda i: (0, 0)),
        out_shape=jax.ShapeDtypeStruct((1, 1), jnp.float32),
    )


def kernel(token_field_values, token_sequence_field_values, labels, fo_token,
           so_token, fo_seq, so_seq, fm_bias, W1, b1, W2, b2):
    tfv = token_field_values.astype(jnp.int32)
    tok_flat = tfv + (jnp.arange(NT, dtype=jnp.int32) * V)[None, :]
    tok_idx = tok_flat.reshape(-1)
    tok_fo_idx = tok_flat.reshape(B // C, C, NT).transpose(0, 2, 1).reshape(-1)

    sq = token_sequence_field_values.astype(jnp.int32)
    sqp = jnp.pad(sq, ((0, 0), (0, 0), (0, LP - L)))
    seq_flat = sqp + (jnp.arange(NS, dtype=jnp.int32) * V)[None, :, None]
    seq_idx = seq_flat.reshape(B // C, C, NS * LP).transpose(0, 2, 1).reshape(-1)

    so_tok_flat = so_token.reshape(NT * V, D)
    so_seq_flat = so_seq.reshape(NS * V, D)
    fo_tok_flat = fo_token.reshape(NT * V)
    fo_seq_flat = fo_seq.reshape(NS * V)

    emb_tok, seq_sum, tok_fo, seq_fo, cnt = _sc_stage()(
        tok_idx, tok_fo_idx, seq_idx, so_tok_flat, so_seq_flat,
        fo_tok_flat, fo_seq_flat)

    bb = 2048
    out = _tc_stage(bb)(
        emb_tok.reshape(B, NT * D),
        seq_sum.reshape(NS, B, D),
        cnt.reshape(NS, B),
        seq_fo.reshape(NS, B),
        tok_fo.reshape(1, B),
        labels.astype(jnp.int32).reshape(1, B),
        so_seq[:, 0, :],
        fo_seq[:, 0].reshape(1, NS),
        W1, b1.reshape(1, INNER), W2, b2.reshape(1, 1),
        fm_bias.reshape(1, 1),
    )
    return out[0, 0]


# speedup vs baseline: 18.9427x; 1.0733x over previous
import functools

import jax
import jax.numpy as jnp
from jax import lax
from jax.experimental import pallas as pl
from jax.experimental.pallas import tpu as pltpu
from jax.experimental.pallas import tpu_sc as plsc

B = 16384
V = 100000
D = 16
NT = 26
NS = 2
L = 50
LP = 64
INNER = 128

NW = 32
CB = B // NW
C = 16
NCHUNK = CB // C
TOKW = C * NT
SEQW = C * NS * LP
HS = SEQW // NS


def _sc_stage():
    mesh = plsc.VectorSubcoreMesh(core_axis_name="c", subcore_axis_name="s")

    vm = pltpu.VMEM
    f32 = jnp.float32
    i32 = jnp.int32

    @functools.partial(
        pl.kernel,
        mesh=mesh,
        compiler_params=pltpu.CompilerParams(use_tc_tiling_on_sc=False),
        out_type=[
            jax.ShapeDtypeStruct((B * NT, D), f32),
            jax.ShapeDtypeStruct((NS * B, D), f32),
            jax.ShapeDtypeStruct((B,), f32),
            jax.ShapeDtypeStruct((NS * B,), f32),
            jax.ShapeDtypeStruct((NS * B,), f32),
        ],
        scratch_types=[
            [vm((TOKW,), i32)] * 2,
            [vm((TOKW,), i32)] * 2,
            [vm((HS,), i32)] * 2,
            [vm((HS,), i32)] * 2,
            [vm((TOKW, D), f32)] * 2,
            [vm((HS, D), f32)] * 2,
            [vm((HS, D), f32)] * 2,
            [vm((TOKW,), f32)] * 2,
            [vm((HS,), f32)] * 2,
            [vm((HS,), f32)] * 2,
            vm((NS * C, D), f32),
            vm((C,), f32),
            vm((NS * C,), f32),
            vm((NS * C,), f32),
            pltpu.VMEM_SHARED((NS * V,), f32),
            [pltpu.SemaphoreType.DMA] * 2,
            [pltpu.SemaphoreType.DMA] * 2,
            [pltpu.SemaphoreType.DMA] * 2,
            [pltpu.SemaphoreType.DMA] * 2,
            [pltpu.SemaphoreType.DMA] * 2,
            [pltpu.SemaphoreType.DMA] * 2,
            [pltpu.SemaphoreType.DMA] * 2,
            pltpu.SemaphoreType.DMA,
        ],
    )
    def k(tok_idx_h, tokfo_idx_h, seq_idx_h, so_tok_h, so_seq_h, fo_tok_h,
          fo_seq_h, emb_tok_h, seq_sum_h, tok_fo_h, seq_fo_h, cnt_h,
          ti, tf, sA, sB, tr, srA, srB, ftv, fsA, fsB,
          seq_sum_v, tok_fo_sv, seq_fo_sv, cnt_sv, sp_fo,
          isem, g_tok, g_sA, g_sB, g_ft, g_fA, g_fB, wsem):
        c_ax = lax.axis_index("c")
        t_ax = lax.axis_index("s")
        wid = t_ax * 2 + c_ax

        def idx_copies(gl, p):
            gg = wid * NCHUNK + gl
            return (
                pltpu.make_async_copy(
                    tok_idx_h.at[pl.ds(gg * TOKW, TOKW)], ti[p], isem[p]),
                pltpu.make_async_copy(
                    tokfo_idx_h.at[pl.ds(gg * TOKW, TOKW)], tf[p], isem[p]),
                pltpu.make_async_copy(
                    seq_idx_h.at[pl.ds(gg * SEQW, HS)], sA[p], isem[p]),
                pltpu.make_async_copy(
                    seq_idx_h.at[pl.ds(gg * SEQW + HS, HS)], sB[p], isem[p]),
            )

        def fire_idx(gl, p):
            for c in idx_copies(gl, p):
                c.start()

        def wait_idx(gl, p):
            for c in idx_copies(gl, p):
                c.wait()

        def gather_copies(p):
            return (
                pltpu.make_async_copy(so_tok_h.at[ti[p]], tr[p], g_tok[p]),
                pltpu.make_async_copy(so_seq_h.at[sA[p]], srA[p], g_sA[p]),
                pltpu.make_async_copy(so_seq_h.at[sB[p]], srB[p], g_sB[p]),
                pltpu.make_async_copy(fo_tok_h.at[tf[p]], ftv[p], g_ft[p]),
                pltpu.make_async_copy(sp_fo.at[sA[p]], fsA[p], g_fA[p]),
                pltpu.make_async_copy(sp_fo.at[sB[p]], fsB[p], g_fB[p]),
            )

        def fire_gathers(p):
            for c in gather_copies(p):
                c.start()

        def wait_gathers(p):
            for c in gather_copies(p):
                c.wait()

        def process(gl, p, k_iter, last_fire_ok):
            gg = wid * NCHUNK + gl
            tok_base = gg * TOKW
            b_base = gg * C

            wait_gathers(p)

            wcopies = [pltpu.make_async_copy(
                tr[p], emb_tok_h.at[pl.ds(tok_base, TOKW)], wsem)]
            wcopies[0].start()

            acc0 = ftv[p][pl.ds(0, C)] + ftv[p][pl.ds(C, C)]
            acc1 = ftv[p][pl.ds(2 * C, C)] + ftv[p][pl.ds(3 * C, C)]
            for f in range(4, NT, 2):
                acc0 = acc0 + ftv[p][pl.ds(f * C, C)]
                acc1 = acc1 + ftv[p][pl.ds((f + 1) * C, C)]
            tok_fo_sv[...] = acc0 + acc1

            for h, (idxb, fob) in enumerate(((sA[p], fsA[p]), (sB[p], fsB[p]))):
                sent = jnp.int32(h * V)
                f0 = fob[pl.ds(0, C)]
                f1 = fob[pl.ds(C, C)]
                c0 = jnp.where(idxb[pl.ds(0, C)] != sent, 1.0, 0.0)
                c1 = jnp.where(idxb[pl.ds(C, C)] != sent, 1.0, 0.0)
                for j in range(2, LP, 2):
                    f0 = f0 + fob[pl.ds(j * C, C)]
                    f1 = f1 + fob[pl.ds((j + 1) * C, C)]
                    c0 = c0 + jnp.where(idxb[pl.ds(j * C, C)] != sent, 1.0, 0.0)
                    c1 = c1 + jnp.where(idxb[pl.ds((j + 1) * C, C)] != sent,
                                        1.0, 0.0)
                seq_fo_sv[pl.ds(h * C, C)] = f0 + f1
                cnt_sv[pl.ds(h * C, C)] = c0 + c1

            if last_fire_ok is not None:
                @pl.when(last_fire_ok)
                def _():
                    fire_idx(gl + 2, p)

            for h, rows in enumerate((srA[p], srB[p])):
                def jstep(j, accs):
                    base = j * C
                    return tuple(accs[b] + rows[base + b] for b in range(C))
                accs = tuple(rows[b] for b in range(C))
                accs = lax.fori_loop(1, LP, jstep, accs)
                for b in range(C):
                    seq_sum_v[h * C + b] = accs[b]

            wcopies.append(pltpu.make_async_copy(
                tok_fo_sv, tok_fo_h.at[pl.ds(b_base, C)], wsem))
            for s in range(NS):
                wcopies.append(pltpu.make_async_copy(
                    seq_fo_sv.at[pl.ds(s * C, C)],
                    seq_fo_h.at[pl.ds(s * B + b_base, C)], wsem))
                wcopies.append(pltpu.make_async_copy(
                    cnt_sv.at[pl.ds(s * C, C)],
                    cnt_h.at[pl.ds(s * B + b_base, C)], wsem))
                wcopies.append(pltpu.make_async_copy(
                    seq_sum_v.at[pl.ds(s * C, C)],
                    seq_sum_h.at[pl.ds(s * B + b_base, C)], wsem))
            for c in wcopies[1:]:
                c.start()
            for c in wcopies:
                c.wait()

        fire_idx(0, 0)
        vslice = NS * V // 8

        @pl.when(t_ax < 8)
        def _():
            pltpu.sync_copy(fo_seq_h.at[pl.ds(t_ax * vslice, vslice)],
                            sp_fo.at[pl.ds(t_ax * vslice, vslice)])

        plsc.subcore_barrier()

        wait_idx(0, 0)
        fire_gathers(0)
        fire_idx(1, 1)

        def body(kk, carry):
            g = 2 * kk
            wait_idx(g + 1, 1)
            fire_gathers(1)
            process(g, 0, kk, kk < (NCHUNK // 2 - 1))
            ok = kk < (NCHUNK // 2 - 1)

            @pl.when(ok)
            def _():
                wait_idx(g + 2, 0)
                fire_gathers(0)

            process(g + 1, 1, kk, ok)
            return carry

        lax.fori_loop(0, NCHUNK // 2, body, 0)

    return k


def _tc_stage(bb):
    grid = (B // bb,)

    def body(emb_ref, ssum_ref, cnt_ref, sfo_ref, tfo_ref, lab_ref, row0_ref,
             fo0_ref, w1_ref, b1_ref, w2_ref, b2_ref, bias_ref, out_ref):
        i = pl.program_id(0)
        et = emb_ref[...]
        ssum = ssum_ref[...]
        cnt = cnt_ref[...]
        sfo = sfo_ref[...]
        row0 = row0_ref[...]
        fo0 = fo0_ref[...]
        npad = jnp.float32(LP) - cnt
        denom = jnp.maximum(cnt, 1.0)
        sagg = (ssum - npad[:, :, None] * row0[:, None, :]) / denom[:, :, None]
        sfo_c = (sfo - npad * fo0.reshape(NS, 1)) / denom
        first = tfo_ref[...][0] + sfo_c.sum(axis=0)

        et3 = et.reshape(bb, NT, D)
        summed = et3.sum(axis=1) + sagg.sum(axis=0)
        sumsq = (et3 * et3).sum(axis=1) + (sagg * sagg).sum(axis=0)
        second = 0.5 * ((summed * summed - sumsq).sum(axis=-1))

        w1 = w1_ref[...]
        h = jnp.dot(et, w1[: NT * D], preferred_element_type=jnp.float32)
        for s in range(NS):
            h = h + jnp.dot(sagg[s], w1[NT * D + s * D: NT * D + (s + 1) * D],
                            preferred_element_type=jnp.float32)
        h = jax.nn.relu(h + b1_ref[...])
        dnn = jnp.dot(h, w2_ref[...], preferred_element_type=jnp.float32)[:, 0]
        dnn = dnn + b2_ref[0, 0]

        logits = bias_ref[0, 0] + first + second + dnn
        y = lab_ref[...][0].astype(jnp.float32)
        bce = (jnp.maximum(logits, 0.0) - logits * y
               + jnp.log1p(jnp.exp(-jnp.abs(logits))))
        part = bce.sum() * (1.0 / B)

        @pl.when(i == 0)
        def _init():
            out_ref[...] = jnp.zeros((1, 1), jnp.float32)

        out_ref[...] += part.reshape(1, 1)

    return pl.pallas_call(
        body,
        grid=grid,
        in_specs=[
            pl.BlockSpec((bb, NT * D), lambda i: (i, 0)),
            pl.BlockSpec((NS, bb, D), lambda i: (0, i, 0)),
            pl.BlockSpec((NS, bb), lambda i: (0, i)),
            pl.BlockSpec((NS, bb), lambda i: (0, i)),
            pl.BlockSpec((1, bb), lambda i: (0, i)),
            pl.BlockSpec((1, bb), lambda i: (0, i)),
            pl.BlockSpec((NS, D), lambda i: (0, 0)),
            pl.BlockSpec((1, NS), lambda i: (0, 0)),
            pl.BlockSpec(((NT + NS) * D, INNER), lambda i: (0, 0)),
            pl.BlockSpec((1, INNER), lambda i: (0, 0)),
            pl.BlockSpec((INNER, 1), lambda i: (0, 0)),
            pl.BlockSpec((1, 1), lambda i: (0, 0)),
            pl.BlockSpec((1, 1), lambda i: (0, 0)),
        ],
        out_specs=pl.BlockSpec((1, 1), lambda i: (0, 0)),
        out_shape=jax.ShapeDtypeStruct((1, 1), jnp.float32),
    )


def kernel(token_field_values, token_sequence_field_values, labels, fo_token,
           so_token, fo_seq, so_seq, fm_bias, W1, b1, W2, b2):
    tfv = token_field_values.astype(jnp.int32)
    tok_flat = tfv + (jnp.arange(NT, dtype=jnp.int32) * V)[None, :]
    tok_idx = tok_flat.reshape(-1)
    tok_fo_idx = tok_flat.reshape(B // C, C, NT).transpose(0, 2, 1).reshape(-1)

    sq = token_sequence_field_values.astype(jnp.int32)
    sqp = jnp.pad(sq, ((0, 0), (0, 0), (0, LP - L)))
    seq_flat = sqp + (jnp.arange(NS, dtype=jnp.int32) * V)[None, :, None]
    seq_idx = seq_flat.reshape(B // C, C, NS * LP).transpose(0, 2, 1).reshape(-1)

    so_tok_flat = so_token.reshape(NT * V, D)
    so_seq_flat = so_seq.reshape(NS * V, D)
    fo_tok_flat = fo_token.reshape(NT * V)
    fo_seq_flat = fo_seq.reshape(NS * V)

    emb_tok, seq_sum, tok_fo, seq_fo, cnt = _sc_stage()(
        tok_idx, tok_fo_idx, seq_idx, so_tok_flat, so_seq_flat,
        fo_tok_flat, fo_seq_flat)

    bb = 2048
    out = _tc_stage(bb)(
        emb_tok.reshape(B, NT * D),
        seq_sum.reshape(NS, B, D),
        cnt.reshape(NS, B),
        seq_fo.reshape(NS, B),
        tok_fo.reshape(1, B),
        labels.astype(jnp.int32).reshape(1, B),
        so_seq[:, 0, :],
        fo_seq[:, 0].reshape(1, NS),
        W1, b1.reshape(1, INNER), W2, b2.reshape(1, 1),
        fm_bias.reshape(1, 1),
    )
    return out[0, 0]
